# baseline (speedup 1.0000x reference)
import jax
from jax import lax
import jax.numpy as jnp
from jax.experimental import pallas as pl
from jax.experimental.pallas import tpu as pltpu
from jax.experimental.pallas import tpu_sc as plsc

B, L = 4096, 50
N = B * L
WORD_VOCAB = 1000000
POS_VOCAB = 1000
WORD_DIM = 64
POS_DIM = 32

NC, NS = 2, 16
NW = NC * NS
PER_W = N // NW
W = 640
N_WIN = PER_W // W
OB = 16


def _sc_gather_packed(word_table, pos_table, rel_table, widx, pidx, ridx):
    mesh = plsc.VectorSubcoreMesh(core_axis_name="c", subcore_axis_name="s")

    @pl.kernel(
        out_type=jax.ShapeDtypeStruct((N, 128), jnp.float32),
        mesh=mesh,
        scratch_types=[
            pltpu.VMEM((W,), jnp.int32),
            pltpu.VMEM((W,), jnp.int32),
            pltpu.VMEM((W,), jnp.int32),
            pltpu.VMEM((W, WORD_DIM), jnp.float32),
            pltpu.VMEM((W, POS_DIM), jnp.float32),
            pltpu.VMEM((W, POS_DIM), jnp.float32),
        ],
        compiler_params=pltpu.CompilerParams(use_tc_tiling_on_sc=False),
    )
    def kern(wt_hbm, pt_hbm, rt_hbm, wi_hbm, pi_hbm, ri_hbm, o_hbm,
             wi_v, pi_v, ri_v, wv, pv, rv):
        wid = lax.axis_index("s") * NC + lax.axis_index("c")
        for w in range(N_WIN):
            base = wid * PER_W + w * W
            pltpu.sync_copy(wi_hbm.at[pl.ds(base, W)], wi_v)
            pltpu.sync_copy(pi_hbm.at[pl.ds(base, W)], pi_v)
            pltpu.sync_copy(ri_hbm.at[pl.ds(base, W)], ri_v)
            pltpu.sync_copy(pt_hbm.at[pi_v], pv)
            pltpu.sync_copy(rt_hbm.at[ri_v], rv)
            pltpu.sync_copy(wt_hbm.at[wi_v], wv)
            pltpu.sync_copy(pv, o_hbm.at[pl.ds(base, W), 0:POS_DIM])
            pltpu.sync_copy(rv, o_hbm.at[pl.ds(base, W),
                                         POS_DIM:2 * POS_DIM])
            pltpu.sync_copy(wv, o_hbm.at[pl.ds(base, W), 2 * POS_DIM:128])

    return kern(word_table, pos_table, rel_table, widx, pidx, ridx)


def _unpack_outputs(packed):
    def body(x_ref, wo_ref, po_ref, ro_ref):
        x = x_ref[...]
        po_ref[...] = x[:, 0:POS_DIM].reshape(OB, L, POS_DIM)
        ro_ref[...] = x[:, POS_DIM:2 * POS_DIM].reshape(OB, L, POS_DIM)
        wo_ref[...] = x[:, 2 * POS_DIM:128].reshape(OB, L, WORD_DIM)

    return pl.pallas_call(
        body,
        grid=(B // OB,),
        in_specs=[pl.BlockSpec((OB * L, 128), lambda i: (i, 0))],
        out_specs=[
            pl.BlockSpec((OB, L, WORD_DIM), lambda i: (i, 0, 0)),
            pl.BlockSpec((OB, L, POS_DIM), lambda i: (i, 0, 0)),
            pl.BlockSpec((OB, L, POS_DIM), lambda i: (i, 0, 0)),
        ],
        out_shape=(
            jax.ShapeDtypeStruct((B, L, WORD_DIM), jnp.float32),
            jax.ShapeDtypeStruct((B, L, POS_DIM), jnp.float32),
            jax.ShapeDtypeStruct((B, L, POS_DIM), jnp.float32),
        ),
        compiler_params=pltpu.CompilerParams(
            dimension_semantics=("parallel",)),
    )(packed)


@jax.jit
def kernel(word_idxs, pos_idxs, rel_idxs, word_table, pos_table, rel_table):
    widx = word_idxs.reshape(N)
    pidx = pos_idxs.reshape(N)
    ridx = rel_idxs.reshape(N)
    packed = _sc_gather_packed(word_table, pos_table, rel_table,
                               widx, pidx, ridx)
    return _unpack_outputs(packed)
